# Initial kernel scaffold; baseline (speedup 1.0000x reference)
import jax
import jax.numpy as jnp
from jax.experimental import pallas as pl


def kernel(boxes, scores):
    raise NotImplementedError("write your pallas kernel here")


# speedup vs baseline: 17.1536x; 17.1536x over previous
import jax
import jax.numpy as jnp
from jax.experimental import pallas as pl
from jax.experimental.pallas import tpu as pltpu

_N = 20000
_NPAD = 20480
_ROWS = 160
_K = 1000
_MAX_DETS = 100
_THR = 0.5
_NEG = float("-inf")


def _nms_body(x1_ref, y1_ref, x2_ref, y2_ref, s_ref, out_ref, sc_ref, ab_ref):
    S = s_ref[...]
    bits = jax.lax.bitcast_convert_type(S, jnp.int32)
    IDX = (jax.lax.broadcasted_iota(jnp.int32, (_ROWS, 128), 0) * 128
           + jax.lax.broadcasted_iota(jnp.int32, (_ROWS, 128), 1))

    def bs1(_, lohi):
        lo, hi = lohi
        mid = lo + (hi - lo) // 2
        cnt = jnp.sum((bits >= mid).astype(jnp.int32))
        ge = cnt >= _K
        return (jnp.where(ge, mid, lo), jnp.where(ge, hi, mid))

    lo, _ = jax.lax.fori_loop(0, 31, bs1, (jnp.int32(0), jnp.int32(0x7F800000)))
    n1 = jnp.sum((bits > lo).astype(jnp.int32))
    m = _K - n1
    tie = bits == lo

    def bs2(_, lohi):
        lo2, hi2 = lohi
        mid = (lo2 + hi2) // 2
        cnt = jnp.sum((tie & (IDX < mid)).astype(jnp.int32))
        ge = cnt >= m
        return (jnp.where(ge, lo2, mid), jnp.where(ge, mid, hi2))

    _, p = jax.lax.fori_loop(0, 15, bs2, (jnp.int32(0), jnp.int32(_NPAD)))
    sel = (bits > lo) | (tie & (IDX < p))
    sc_ref[...] = jnp.where(sel, S, _NEG)

    X1 = x1_ref[...]
    Y1 = y1_ref[...]
    X2 = x2_ref[...]
    Y2 = y2_ref[...]
    ab_ref[...] = (X2 - X1) * (Y2 - Y1)

    def nms(i, out):
        Sv = sc_ref[...]
        M = jnp.max(Sv)
        valid = M > _NEG
        idx = jnp.min(jnp.where(Sv == M, IDX, jnp.int32(0x7FFFFFFF)))
        em = IDX == idx
        emf = em.astype(jnp.float32)
        bx1 = jnp.sum(x1_ref[...] * emf)
        by1 = jnp.sum(y1_ref[...] * emf)
        bx2 = jnp.sum(x2_ref[...] * emf)
        by2 = jnp.sum(y2_ref[...] * emf)
        xx1 = jnp.maximum(bx1, x1_ref[...])
        yy1 = jnp.maximum(by1, y1_ref[...])
        xx2 = jnp.minimum(bx2, x2_ref[...])
        yy2 = jnp.minimum(by2, y2_ref[...])
        inter = jnp.maximum(xx2 - xx1, 0.0) * jnp.maximum(yy2 - yy1, 0.0)
        area_a = (bx2 - bx1) * (by2 - by1)
        union = area_a + ab_ref[...] - inter
        iou = inter / jnp.maximum(union, 1e-9)
        sc_ref[...] = jnp.where((iou >= _THR) | em, _NEG, Sv)

        row = jax.lax.broadcasted_iota(jnp.int32, (128, 8), 0)
        lane = jax.lax.broadcasted_iota(jnp.int32, (128, 8), 1)
        z = jnp.float32(0.0)
        vals = (jnp.where(lane == 0, jnp.where(valid, bx1, z), z)
                + jnp.where(lane == 1, jnp.where(valid, by1, z), z)
                + jnp.where(lane == 2, jnp.where(valid, bx2, z), z)
                + jnp.where(lane == 3, jnp.where(valid, by2, z), z)
                + jnp.where(lane == 4, jnp.where(valid, M, z), z))
        return jnp.where(row == i, vals, out)

    out_ref[...] = jax.lax.fori_loop(
        0, _MAX_DETS, nms, jnp.zeros((128, 8), jnp.float32))


def kernel(boxes, scores):
    b = jnp.pad(boxes, ((0, _NPAD - _N), (0, 0)))
    s = jnp.pad(scores, (0, _NPAD - _N), constant_values=-1.0)
    x1 = b[:, 0].reshape(_ROWS, 128)
    y1 = b[:, 1].reshape(_ROWS, 128)
    x2 = b[:, 2].reshape(_ROWS, 128)
    y2 = b[:, 3].reshape(_ROWS, 128)
    out = pl.pallas_call(
        _nms_body,
        out_shape=jax.ShapeDtypeStruct((128, 8), jnp.float32),
        scratch_shapes=[
            pltpu.VMEM((_ROWS, 128), jnp.float32),
            pltpu.VMEM((_ROWS, 128), jnp.float32),
        ],
    )(x1, y1, x2, y2, s.reshape(_ROWS, 128))
    return out[:_MAX_DETS, :5]


# speedup vs baseline: 17.6711x; 1.0302x over previous
import jax
import jax.numpy as jnp
from jax.experimental import pallas as pl
from jax.experimental.pallas import tpu as pltpu

_N = 20000
_NPAD = 20480
_ROWS = 160
_K = 1000
_MAX_DETS = 100
_THR = 0.5
_NEG = float("-inf")


def _nms_body(x1_ref, y1_ref, x2_ref, y2_ref, s_ref, out_ref, sc_ref, ab_ref):
    S = s_ref[...]
    bits = jax.lax.bitcast_convert_type(S, jnp.int32)
    IDX = (jax.lax.broadcasted_iota(jnp.int32, (_ROWS, 128), 0) * 128
           + jax.lax.broadcasted_iota(jnp.int32, (_ROWS, 128), 1))

    def bs1(_, lohi):
        lo, hi = lohi
        mid = lo + (hi - lo) // 2
        cnt = jnp.sum((bits >= mid).astype(jnp.int32))
        ge = cnt >= _K
        return (jnp.where(ge, mid, lo), jnp.where(ge, hi, mid))

    lo, _ = jax.lax.fori_loop(0, 31, bs1, (jnp.int32(0), jnp.int32(0x7F800000)))
    n1 = jnp.sum((bits > lo).astype(jnp.int32))
    m = _K - n1
    tie = bits == lo

    def bs2(_, lohi):
        lo2, hi2 = lohi
        mid = (lo2 + hi2) // 2
        cnt = jnp.sum((tie & (IDX < mid)).astype(jnp.int32))
        ge = cnt >= m
        return (jnp.where(ge, lo2, mid), jnp.where(ge, mid, hi2))

    _, p = jax.lax.fori_loop(0, 15, bs2, (jnp.int32(0), jnp.int32(_NPAD)))
    sel = (bits > lo) | (tie & (IDX < p))
    sc_ref[...] = jnp.where(sel, S, _NEG)

    X1 = x1_ref[...]
    Y1 = y1_ref[...]
    X2 = x2_ref[...]
    Y2 = y2_ref[...]
    ab_ref[...] = (X2 - X1) * (Y2 - Y1)

    def nms(i, out):
        Sv = sc_ref[...]
        M = jnp.max(Sv)
        valid = M > _NEG
        idx = jnp.min(jnp.where(Sv == M, IDX, jnp.int32(0x7FFFFFFF)))
        r = idx // 128
        c = idx % 128
        lm = jax.lax.broadcasted_iota(jnp.int32, (1, 128), 1) == c
        bx1 = jnp.sum(jnp.where(lm, x1_ref[pl.ds(r, 1), :], 0.0))
        by1 = jnp.sum(jnp.where(lm, y1_ref[pl.ds(r, 1), :], 0.0))
        bx2 = jnp.sum(jnp.where(lm, x2_ref[pl.ds(r, 1), :], 0.0))
        by2 = jnp.sum(jnp.where(lm, y2_ref[pl.ds(r, 1), :], 0.0))
        xx1 = jnp.maximum(bx1, x1_ref[...])
        yy1 = jnp.maximum(by1, y1_ref[...])
        xx2 = jnp.minimum(bx2, x2_ref[...])
        yy2 = jnp.minimum(by2, y2_ref[...])
        inter = jnp.maximum(xx2 - xx1, 0.0) * jnp.maximum(yy2 - yy1, 0.0)
        area_a = (bx2 - bx1) * (by2 - by1)
        union = area_a + ab_ref[...] - inter
        iou = inter / jnp.maximum(union, 1e-9)
        sc_ref[...] = jnp.where(iou >= _THR, _NEG, Sv)
        sc_ref[pl.ds(r, 1), :] = jnp.where(lm, _NEG, sc_ref[pl.ds(r, 1), :])

        row = jax.lax.broadcasted_iota(jnp.int32, (128, 8), 0)
        lane = jax.lax.broadcasted_iota(jnp.int32, (128, 8), 1)
        z = jnp.float32(0.0)
        vals = (jnp.where(lane == 0, jnp.where(valid, bx1, z), z)
                + jnp.where(lane == 1, jnp.where(valid, by1, z), z)
                + jnp.where(lane == 2, jnp.where(valid, bx2, z), z)
                + jnp.where(lane == 3, jnp.where(valid, by2, z), z)
                + jnp.where(lane == 4, jnp.where(valid, M, z), z))
        return jnp.where(row == i, vals, out)

    out_ref[...] = jax.lax.fori_loop(
        0, _MAX_DETS, nms, jnp.zeros((128, 8), jnp.float32))


def kernel(boxes, scores):
    b = jnp.pad(boxes, ((0, _NPAD - _N), (0, 0)))
    s = jnp.pad(scores, (0, _NPAD - _N), constant_values=-1.0)
    x1 = b[:, 0].reshape(_ROWS, 128)
    y1 = b[:, 1].reshape(_ROWS, 128)
    x2 = b[:, 2].reshape(_ROWS, 128)
    y2 = b[:, 3].reshape(_ROWS, 128)
    out = pl.pallas_call(
        _nms_body,
        out_shape=jax.ShapeDtypeStruct((128, 8), jnp.float32),
        scratch_shapes=[
            pltpu.VMEM((_ROWS, 128), jnp.float32),
            pltpu.VMEM((_ROWS, 128), jnp.float32),
        ],
    )(x1, y1, x2, y2, s.reshape(_ROWS, 128))
    return out[:_MAX_DETS, :5]
